# baseline (speedup 1.0000x reference)
import functools

import jax
import jax.numpy as jnp
from jax.experimental import pallas as pl
from jax.experimental.pallas import tpu as pltpu

N = 10000
D_IN = 128
D_OUT = 128
BLK = 200


def _decoder_kernel(x_ref, w_ref, b_ref, a0_ref, a1_ref, out_ref,
                    support, gcn, m, *, nsteps):
    i = pl.program_id(0)

    @pl.when(i == 0)
    def _init():
        support[...] = jnp.dot(x_ref[...], w_ref[...],
                               preferred_element_type=jnp.float32)
        m[...] = jnp.full_like(m[...], -jnp.inf)

    g0 = jnp.dot(a0_ref[...], support[...],
                 preferred_element_type=jnp.float32) + b_ref[...]
    g1 = jnp.dot(a1_ref[...], support[...],
                 preferred_element_type=jnp.float32) + b_ref[...]
    gcn[pl.ds(2 * i * BLK, BLK), :] = g0
    gcn[pl.ds((2 * i + 1) * BLK, BLK), :] = g1
    gmax = jnp.maximum(jnp.max(g0, axis=0, keepdims=True),
                       jnp.max(g1, axis=0, keepdims=True))
    m[...] = jnp.maximum(m[...], gmax)

    @pl.when(i == nsteps - 1)
    def _flush():
        e = jnp.exp(gcn[...] - m[...])
        z = jnp.sum(e, axis=0, keepdims=True)
        acc = jax.lax.dot_general(e, x_ref[...], (((0,), (0,)), ((), ())),
                                  preferred_element_type=jnp.float32)
        out_ref[...] = acc / z.T


def kernel(main_feat, main_adj, W, b):
    nsteps = N // (2 * BLK)
    b2d = b.reshape(1, D_OUT)
    out = pl.pallas_call(
        functools.partial(_decoder_kernel, nsteps=nsteps),
        grid=(nsteps,),
        in_specs=[
            pl.BlockSpec((N, D_IN), lambda i: (0, 0)),
            pl.BlockSpec((D_IN, D_OUT), lambda i: (0, 0)),
            pl.BlockSpec((1, D_OUT), lambda i: (0, 0)),
            pl.BlockSpec((BLK, N), lambda i: (2 * i, 0)),
            pl.BlockSpec((BLK, N), lambda i: (2 * i + 1, 0)),
        ],
        out_specs=pl.BlockSpec((D_OUT, D_IN), lambda i: (0, 0)),
        out_shape=jax.ShapeDtypeStruct((D_OUT, D_IN), jnp.float32),
        scratch_shapes=[
            pltpu.VMEM((N, D_OUT), jnp.float32),
            pltpu.VMEM((N, D_OUT), jnp.float32),
            pltpu.VMEM((1, D_OUT), jnp.float32),
        ],
        compiler_params=pltpu.CompilerParams(
            dimension_semantics=("arbitrary",),
        ),
    )(main_feat, W, b2d, main_adj, main_adj)
    return out


# speedup vs baseline: 1.0158x; 1.0158x over previous
import functools

import jax
import jax.numpy as jnp
from jax.experimental import pallas as pl
from jax.experimental.pallas import tpu as pltpu

N = 10000
D_IN = 128
D_OUT = 128
BLK = 200


def _decoder_kernel(x_ref, w_ref, b_ref, a_ref, out_ref,
                    support, gcn, m, m_old, z, acc, *, nsteps):
    i = pl.program_id(0)

    @pl.when(i == 0)
    def _init():
        support[...] = jnp.dot(x_ref[...], w_ref[...],
                               preferred_element_type=jnp.float32)
        m[...] = jnp.full_like(m[...], -jnp.inf)

    g = jnp.dot(a_ref[...], support[...],
                preferred_element_type=jnp.float32) + b_ref[...]
    gcn[pl.ds(i * BLK, BLK), :] = g
    m[...] = jnp.maximum(m[...], jnp.max(g, axis=0, keepdims=True))

    @pl.when(i == nsteps - 2)
    def _partial_flush():
        m_old[...] = m[...]
        e = jnp.exp(gcn[: N - BLK, :] - m[...])
        z[...] = jnp.sum(e, axis=0, keepdims=True)
        acc[...] = jax.lax.dot_general(
            e, x_ref[: N - BLK, :], (((0,), (0,)), ((), ())),
            preferred_element_type=jnp.float32)

    @pl.when(i == nsteps - 1)
    def _flush():
        g_last = gcn[pl.ds(N - BLK, BLK), :]
        m_new = m[...]
        e_last = jnp.exp(g_last - m_new)
        upd = jax.lax.dot_general(
            e_last, x_ref[pl.ds(N - BLK, BLK), :], (((0,), (0,)), ((), ())),
            preferred_element_type=jnp.float32)
        alpha = jnp.exp(m_old[...] - m_new)
        z_tot = z[...] * alpha + jnp.sum(e_last, axis=0, keepdims=True)
        out_ref[...] = (acc[...] * alpha.T + upd) / z_tot.T


def kernel(main_feat, main_adj, W, b):
    nsteps = N // BLK
    b2d = b.reshape(1, D_OUT)
    out = pl.pallas_call(
        functools.partial(_decoder_kernel, nsteps=nsteps),
        grid=(nsteps,),
        in_specs=[
            pl.BlockSpec((N, D_IN), lambda i: (0, 0)),
            pl.BlockSpec((D_IN, D_OUT), lambda i: (0, 0)),
            pl.BlockSpec((1, D_OUT), lambda i: (0, 0)),
            pl.BlockSpec((BLK, N), lambda i: (i, 0)),
        ],
        out_specs=pl.BlockSpec((D_OUT, D_IN), lambda i: (0, 0)),
        out_shape=jax.ShapeDtypeStruct((D_OUT, D_IN), jnp.float32),
        scratch_shapes=[
            pltpu.VMEM((N, D_OUT), jnp.float32),
            pltpu.VMEM((N, D_OUT), jnp.float32),
            pltpu.VMEM((1, D_OUT), jnp.float32),
            pltpu.VMEM((1, D_OUT), jnp.float32),
            pltpu.VMEM((1, D_OUT), jnp.float32),
            pltpu.VMEM((D_OUT, D_IN), jnp.float32),
        ],
        compiler_params=pltpu.CompilerParams(
            dimension_semantics=("arbitrary",),
        ),
    )(main_feat, W, b2d, main_adj)
    return out
